# Initial kernel scaffold; baseline (speedup 1.0000x reference)
import jax
import jax.numpy as jnp
from jax.experimental import pallas as pl


def kernel(x, slow_W, out_W, ln_g, ln_b):
    raise NotImplementedError("write your pallas kernel here")


# speedup vs baseline: 10.3567x; 10.3567x over previous
import functools

import jax
import jax.numpy as jnp
from jax.experimental import pallas as pl
from jax.experimental.pallas import tpu as pltpu

S, B, D, H, DH = 512, 32, 256, 8, 32
E = 5 * DH + 2
G = B * H
GC = G // 2
LN_EPS = 1e-5

T_BLK = 64
ROW_BLK = 256


def _ln_qkv_body(x_ref, w_ref, g_ref, b_ref, out_ref):
    x = x_ref[...]
    mu = jnp.mean(x, axis=1, keepdims=True)
    xc = x - mu
    var = jnp.mean(xc * xc, axis=1, keepdims=True)
    o = xc * jax.lax.rsqrt(var + LN_EPS) * g_ref[...] + b_ref[...]
    out_ref[...] = jnp.dot(o, w_ref[...], preferred_element_type=jnp.float32)


def _out_body(h_ref, w_ref, x_ref, out_ref):
    out_ref[...] = x_ref[...] + jnp.dot(
        h_ref[...], w_ref[...], preferred_element_type=jnp.float32)


def _scan_body(q_ref, k_ref, v_ref, rk_ref, rv_ref, be_ref, rbe_ref,
               out_ref,
               W_s, R_s, h_s, qa_s, ka_s, rka_s, bb_s, rbb_s):
    sb = pl.program_id(1)

    @pl.when(sb == 0)
    def _init():
        W_s[...] = jnp.zeros_like(W_s)
        R_s[...] = jnp.zeros_like(R_s)
        h_s[...] = jnp.zeros_like(h_s)

    q = q_ref[...]
    qa = jnp.where(q > 0, q + 1.0, jnp.exp(q))
    qa_s[...] = qa / jnp.sum(qa, axis=1, keepdims=True)
    k = k_ref[...]
    ka = jnp.where(k > 0, k + 1.0, jnp.exp(k))
    ka_s[...] = ka / jnp.sum(ka, axis=1, keepdims=True)
    rk = rk_ref[...]
    rk = rk - jnp.max(rk, axis=1, keepdims=True)
    erk = jnp.exp(rk)
    rka_s[...] = erk / jnp.sum(erk, axis=1, keepdims=True)
    bb_s[...] = jnp.broadcast_to(
        jax.nn.sigmoid(be_ref[...])[:, None, :], (T_BLK, DH, GC))
    rbb_s[...] = jnp.broadcast_to(
        jax.nn.sigmoid(rbe_ref[...])[:, None, :], (T_BLK, DH, GC))

    def step(t, carry):
        W, R, h = carry
        qt = qa_s[t]
        kt = ka_s[t]
        rkt = rka_s[t]
        vt = v_ref[t]
        rvt = rv_ref[t]
        bt = bb_s[t]
        rbt = rbb_s[t]

        v_old = jnp.sum(W * kt[None, :, :], axis=1)
        d = bt * (vt - v_old)
        W = W + d[:, None, :] * kt[None, :, :]
        z = jnp.sum(W * qt[None, :, :], axis=1)

        m = jnp.max(h, axis=0, keepdims=True)
        eh = jnp.exp(h - m)
        qr = eh / jnp.sum(eh, axis=0, keepdims=True)
        v_old_r = jnp.sum(R * rkt[None, :, :], axis=1)
        dr = rbt * (rvt - v_old_r)
        R = R + dr[:, None, :] * rkt[None, :, :]
        h = z + jnp.sum(R * qr[None, :, :], axis=1)

        out_ref[t] = h
        return W, R, h

    W, R, h = jax.lax.fori_loop(
        0, T_BLK, step, (W_s[...], R_s[...], h_s[...]))
    W_s[...] = W
    R_s[...] = R
    h_s[...] = h


def kernel(x, slow_W, out_W, ln_g, ln_b):
    x2d = x.reshape(S * B, D)

    qkv = pl.pallas_call(
        _ln_qkv_body,
        grid=(S * B // ROW_BLK,),
        in_specs=[
            pl.BlockSpec((ROW_BLK, D), lambda i: (i, 0)),
            pl.BlockSpec((D, H * E), lambda i: (0, 0)),
            pl.BlockSpec((1, D), lambda i: (0, 0)),
            pl.BlockSpec((1, D), lambda i: (0, 0)),
        ],
        out_specs=pl.BlockSpec((ROW_BLK, H * E), lambda i: (i, 0)),
        out_shape=jax.ShapeDtypeStruct((S * B, H * E), jnp.float32),
        compiler_params=pltpu.CompilerParams(
            dimension_semantics=("parallel",)),
    )(x2d, slow_W.T, ln_g.reshape(1, D), ln_b.reshape(1, D))

    qkv_t = qkv.reshape(S, B, H, E).transpose(0, 3, 2, 1).reshape(S, E, G)
    q = qkv_t[:, 0 * DH:1 * DH, :]
    k = qkv_t[:, 1 * DH:2 * DH, :]
    v = qkv_t[:, 2 * DH:3 * DH, :]
    rk = qkv_t[:, 3 * DH:4 * DH, :]
    rv = qkv_t[:, 4 * DH:5 * DH, :]
    beta = qkv_t[:, 5 * DH, :]
    rbeta = qkv_t[:, 5 * DH + 1, :]

    vec_spec = pl.BlockSpec((T_BLK, DH, GC), lambda c, s: (s, 0, c))
    sc_spec = pl.BlockSpec((T_BLK, GC), lambda c, s: (s, c))
    f32 = jnp.float32
    hs = pl.pallas_call(
        _scan_body,
        grid=(2, S // T_BLK),
        in_specs=[vec_spec] * 5 + [sc_spec] * 2,
        out_specs=vec_spec,
        out_shape=jax.ShapeDtypeStruct((S, DH, G), f32),
        scratch_shapes=[
            pltpu.VMEM((DH, DH, GC), f32),
            pltpu.VMEM((DH, DH, GC), f32),
            pltpu.VMEM((DH, GC), f32),
            pltpu.VMEM((T_BLK, DH, GC), f32),
            pltpu.VMEM((T_BLK, DH, GC), f32),
            pltpu.VMEM((T_BLK, DH, GC), f32),
            pltpu.VMEM((T_BLK, DH, GC), f32),
            pltpu.VMEM((T_BLK, DH, GC), f32),
        ],
        compiler_params=pltpu.CompilerParams(
            dimension_semantics=("parallel", "arbitrary")),
    )(q, k, v, rk, rv, beta, rbeta)

    hs2d = (hs.reshape(S, DH, H, B).transpose(0, 3, 2, 1)
            .reshape(S * B, H * DH))

    y = pl.pallas_call(
        _out_body,
        grid=(S * B // ROW_BLK,),
        in_specs=[
            pl.BlockSpec((ROW_BLK, H * DH), lambda i: (i, 0)),
            pl.BlockSpec((H * DH, D), lambda i: (0, 0)),
            pl.BlockSpec((ROW_BLK, D), lambda i: (i, 0)),
        ],
        out_specs=pl.BlockSpec((ROW_BLK, D), lambda i: (i, 0)),
        out_shape=jax.ShapeDtypeStruct((S * B, D), jnp.float32),
        compiler_params=pltpu.CompilerParams(
            dimension_semantics=("parallel",)),
    )(hs2d, out_W.T, x2d)

    return y.reshape(S, B, D)


# speedup vs baseline: 11.9254x; 1.1515x over previous
import functools

import jax
import jax.numpy as jnp
from jax.experimental import pallas as pl
from jax.experimental.pallas import tpu as pltpu

S, B, D, H, DH = 512, 32, 256, 8, 32
E = 5 * DH + 2
G = B * H
GC = G // 2
LN_EPS = 1e-5

T_BLK = 64
ROW_BLK = 256


def _ln_qkv_body(x_ref, w_ref, g_ref, b_ref, out_ref):
    x = x_ref[...]
    mu = jnp.mean(x, axis=1, keepdims=True)
    xc = x - mu
    var = jnp.mean(xc * xc, axis=1, keepdims=True)
    o = xc * jax.lax.rsqrt(var + LN_EPS) * g_ref[...] + b_ref[...]
    out_ref[...] = jnp.dot(o, w_ref[...], preferred_element_type=jnp.float32)


def _out_body(h_ref, w_ref, x_ref, out_ref):
    out_ref[...] = x_ref[...] + jnp.dot(
        h_ref[...], w_ref[...], preferred_element_type=jnp.float32)


def _scan_body(q_ref, k_ref, v_ref, rk_ref, rv_ref, be_ref, rbe_ref,
               out_ref,
               W_s, R_s, h_s, qa_s, ka_s, rka_s, bb_s, rbb_s):
    sb = pl.program_id(1)

    @pl.when(sb == 0)
    def _init():
        W_s[...] = jnp.zeros_like(W_s)
        R_s[...] = jnp.zeros_like(R_s)
        h_s[...] = jnp.zeros_like(h_s)

    q = q_ref[...]
    qa = jnp.where(q > 0, q + 1.0, jnp.exp(q))
    qa_s[...] = qa / jnp.sum(qa, axis=1, keepdims=True)
    k = k_ref[...]
    ka = jnp.where(k > 0, k + 1.0, jnp.exp(k))
    ka_s[...] = ka / jnp.sum(ka, axis=1, keepdims=True)
    rk = rk_ref[...]
    rk = rk - jnp.max(rk, axis=1, keepdims=True)
    erk = jnp.exp(rk)
    rka_s[...] = erk / jnp.sum(erk, axis=1, keepdims=True)
    bb_s[...] = jnp.broadcast_to(
        jax.nn.sigmoid(be_ref[...])[:, None, :], (T_BLK, 8, GC))
    rbb_s[...] = jnp.broadcast_to(
        jax.nn.sigmoid(rbe_ref[...])[:, None, :], (T_BLK, 8, GC))

    def step(t, h):
        k8 = jnp.broadcast_to(ka_s[t][:, None, :], (DH, 8, GC))
        q8 = jnp.broadcast_to(qa_s[t][:, None, :], (DH, 8, GC))

        Wv = W_s[...].reshape(DH, DH // 8, 8, GC)
        v_old = jnp.sum(Wv * k8[:, None], axis=0)
        z_old = jnp.sum(Wv * q8[:, None], axis=0)
        vt = v_ref[t].reshape(DH // 8, 8, GC)
        bt = bb_s[t][None]
        d = bt * (vt - v_old)
        W_s[...] = (Wv + k8[:, None] * d[None]).reshape(DH, DH, GC)
        kq = jnp.sum(k8 * q8, axis=0)
        z = z_old + d * kq[None]

        m = jnp.max(h, axis=(0, 1), keepdims=True)
        eh = jnp.exp(h - m)
        qr = eh / jnp.sum(eh, axis=(0, 1), keepdims=True)
        qr8 = jnp.broadcast_to(qr.reshape(DH, 1, GC), (DH, 8, GC))
        rk8 = jnp.broadcast_to(rka_s[t][:, None, :], (DH, 8, GC))

        Rv = R_s[...].reshape(DH, DH // 8, 8, GC)
        v_old_r = jnp.sum(Rv * rk8[:, None], axis=0)
        h_old = jnp.sum(Rv * qr8[:, None], axis=0)
        rvt = rv_ref[t].reshape(DH // 8, 8, GC)
        rbt = rbb_s[t][None]
        dr = rbt * (rvt - v_old_r)
        R_s[...] = (Rv + rk8[:, None] * dr[None]).reshape(DH, DH, GC)
        rkq = jnp.sum(rk8 * qr8, axis=0)
        h = z + h_old + dr * rkq[None]

        out_ref[t] = h.reshape(DH, GC)
        return h

    h = jax.lax.fori_loop(
        0, T_BLK, step, h_s[...].reshape(DH // 8, 8, GC))
    h_s[...] = h.reshape(DH, GC)


def kernel(x, slow_W, out_W, ln_g, ln_b):
    x2d = x.reshape(S * B, D)

    qkv = pl.pallas_call(
        _ln_qkv_body,
        grid=(S * B // ROW_BLK,),
        in_specs=[
            pl.BlockSpec((ROW_BLK, D), lambda i: (i, 0)),
            pl.BlockSpec((D, H * E), lambda i: (0, 0)),
            pl.BlockSpec((1, D), lambda i: (0, 0)),
            pl.BlockSpec((1, D), lambda i: (0, 0)),
        ],
        out_specs=pl.BlockSpec((ROW_BLK, H * E), lambda i: (i, 0)),
        out_shape=jax.ShapeDtypeStruct((S * B, H * E), jnp.float32),
        compiler_params=pltpu.CompilerParams(
            dimension_semantics=("parallel",)),
    )(x2d, slow_W.T, ln_g.reshape(1, D), ln_b.reshape(1, D))

    qkv_t = qkv.reshape(S, B, H, E).transpose(0, 3, 2, 1).reshape(S, E, G)
    q = qkv_t[:, 0 * DH:1 * DH, :]
    k = qkv_t[:, 1 * DH:2 * DH, :]
    v = qkv_t[:, 2 * DH:3 * DH, :]
    rk = qkv_t[:, 3 * DH:4 * DH, :]
    rv = qkv_t[:, 4 * DH:5 * DH, :]
    beta = qkv_t[:, 5 * DH, :]
    rbeta = qkv_t[:, 5 * DH + 1, :]

    vec_spec = pl.BlockSpec((T_BLK, DH, GC), lambda c, s: (s, 0, c))
    sc_spec = pl.BlockSpec((T_BLK, GC), lambda c, s: (s, c))
    f32 = jnp.float32
    hs = pl.pallas_call(
        _scan_body,
        grid=(2, S // T_BLK),
        in_specs=[vec_spec] * 5 + [sc_spec] * 2,
        out_specs=vec_spec,
        out_shape=jax.ShapeDtypeStruct((S, DH, G), f32),
        scratch_shapes=[
            pltpu.VMEM((DH, DH, GC), f32),
            pltpu.VMEM((DH, DH, GC), f32),
            pltpu.VMEM((DH, GC), f32),
            pltpu.VMEM((T_BLK, DH, GC), f32),
            pltpu.VMEM((T_BLK, DH, GC), f32),
            pltpu.VMEM((T_BLK, DH, GC), f32),
            pltpu.VMEM((T_BLK, 8, GC), f32),
            pltpu.VMEM((T_BLK, 8, GC), f32),
        ],
        compiler_params=pltpu.CompilerParams(
            dimension_semantics=("parallel", "arbitrary")),
    )(q, k, v, rk, rv, beta, rbeta)

    hs2d = (hs.reshape(S, DH, H, B).transpose(0, 3, 2, 1)
            .reshape(S * B, H * DH))

    y = pl.pallas_call(
        _out_body,
        grid=(S * B // ROW_BLK,),
        in_specs=[
            pl.BlockSpec((ROW_BLK, H * DH), lambda i: (i, 0)),
            pl.BlockSpec((H * DH, D), lambda i: (0, 0)),
            pl.BlockSpec((ROW_BLK, D), lambda i: (i, 0)),
        ],
        out_specs=pl.BlockSpec((ROW_BLK, D), lambda i: (i, 0)),
        out_shape=jax.ShapeDtypeStruct((S * B, D), jnp.float32),
        compiler_params=pltpu.CompilerParams(
            dimension_semantics=("parallel",)),
    )(hs2d, out_W.T, x2d)

    return y.reshape(S, B, D)


# speedup vs baseline: 14.4616x; 1.2127x over previous
import functools

import jax
import jax.numpy as jnp
from jax.experimental import pallas as pl
from jax.experimental.pallas import tpu as pltpu

S, B, D, H, DH = 512, 32, 256, 8, 32
E = 5 * DH + 2
G = B * H
GC = G // 2
LN_EPS = 1e-5

T_BLK = 64
ROW_BLK = 256


def _ln_qkv_body(x_ref, w_ref, g_ref, b_ref, out_ref):
    x = x_ref[...]
    mu = jnp.mean(x, axis=1, keepdims=True)
    xc = x - mu
    var = jnp.mean(xc * xc, axis=1, keepdims=True)
    o = xc * jax.lax.rsqrt(var + LN_EPS) * g_ref[...] + b_ref[...]
    out_ref[...] = jnp.dot(o, w_ref[...], preferred_element_type=jnp.float32)


def _out_body(h_ref, w_ref, x_ref, out_ref):
    out_ref[...] = x_ref[...] + jnp.dot(
        h_ref[...], w_ref[...], preferred_element_type=jnp.float32)


def _rep8(x2d):
    return jnp.broadcast_to(x2d.reshape(DH, 1, GC), (DH, 8, GC))


def _dot_rep(a2d, b2d):
    p = (a2d * b2d).reshape(DH // 8, 8, GC)
    s = p[0] + p[1] + p[2] + p[3]
    tot = jnp.sum(s, axis=0, keepdims=True)
    return jnp.broadcast_to(tot, (8, GC))


def _scan_body(qkv_ref, out_ref,
               W_s, R_s, h_s, qa_s, ka_s, rka_s, bb_s, rbb_s,
               kp_s, rkp_s, dp_s, drp_s):
    sb = pl.program_id(1)

    @pl.when(sb == 0)
    def _init():
        W_s[...] = jnp.zeros_like(W_s)
        R_s[...] = jnp.zeros_like(R_s)
        h_s[...] = jnp.zeros_like(h_s)
        kp_s[...] = jnp.zeros_like(kp_s)
        rkp_s[...] = jnp.zeros_like(rkp_s)
        dp_s[...] = jnp.zeros_like(dp_s)
        drp_s[...] = jnp.zeros_like(drp_s)

    q = qkv_ref[:, 0 * DH:1 * DH, :]
    qa = jnp.where(q > 0, q + 1.0, jnp.exp(q))
    qa_s[...] = qa / jnp.sum(qa, axis=1, keepdims=True)
    k = qkv_ref[:, 1 * DH:2 * DH, :]
    ka = jnp.where(k > 0, k + 1.0, jnp.exp(k))
    ka_s[...] = ka / jnp.sum(ka, axis=1, keepdims=True)
    rk = qkv_ref[:, 3 * DH:4 * DH, :]
    rk = rk - jnp.max(rk, axis=1, keepdims=True)
    erk = jnp.exp(rk)
    rka_s[...] = erk / jnp.sum(erk, axis=1, keepdims=True)
    bb_s[...] = jnp.broadcast_to(
        jax.nn.sigmoid(qkv_ref[:, 5 * DH, :])[:, None, :], (T_BLK, 8, GC))
    rbb_s[...] = jnp.broadcast_to(
        jax.nn.sigmoid(qkv_ref[:, 5 * DH + 1, :])[:, None, :],
        (T_BLK, 8, GC))

    def step(t, carry):
        h, kp, rkp, dp, drp = carry
        kt = ka_s[t]
        qt = qa_s[t]
        k8 = _rep8(kt)
        q8 = _rep8(qt)

        accv = [jnp.zeros((DH // 8, 8, GC), jnp.float32)] * 2
        accz = [jnp.zeros((DH // 8, 8, GC), jnp.float32)] * 2
        for j in range(DH):
            w = W_s[j] + kp[j][None] * dp
            W_s[j] = w
            accv[j % 2] = accv[j % 2] + w * k8[j][None]
            accz[j % 2] = accz[j % 2] + w * q8[j][None]
        v_old = accv[0] + accv[1]
        z_old = accz[0] + accz[1]

        vt = qkv_ref[t, 2 * DH:3 * DH, :].reshape(DH // 8, 8, GC)
        bt = bb_s[t][None]
        d = bt * (vt - v_old)
        kq = _dot_rep(kt, qt)
        z = z_old + d * kq[None]

        m = jnp.max(h, axis=(0, 1), keepdims=True)
        eh = jnp.exp(h - m)
        qr = eh / jnp.sum(eh, axis=(0, 1), keepdims=True)
        qr2 = qr.reshape(DH, GC)
        qr8 = _rep8(qr2)
        rkt = rka_s[t]
        rk8 = _rep8(rkt)

        accvr = [jnp.zeros((DH // 8, 8, GC), jnp.float32)] * 2
        acch = [jnp.zeros((DH // 8, 8, GC), jnp.float32)] * 2
        for j in range(DH):
            r = R_s[j] + rkp[j][None] * drp
            R_s[j] = r
            accvr[j % 2] = accvr[j % 2] + r * rk8[j][None]
            acch[j % 2] = acch[j % 2] + r * qr8[j][None]
        v_old_r = accvr[0] + accvr[1]
        h_old = acch[0] + acch[1]

        rvt = qkv_ref[t, 4 * DH:5 * DH, :].reshape(DH // 8, 8, GC)
        rbt = rbb_s[t][None]
        dr = rbt * (rvt - v_old_r)
        rkq = _dot_rep(rkt, qr2)
        h = z + h_old + dr * rkq[None]

        out_ref[t] = h.reshape(DH, GC)
        return h, k8, rk8, d, dr

    h, kp, rkp, dp, drp = jax.lax.fori_loop(
        0, T_BLK, step,
        (h_s[...].reshape(DH // 8, 8, GC), kp_s[...], rkp_s[...],
         dp_s[...], drp_s[...]))
    h_s[...] = h.reshape(DH, GC)
    kp_s[...] = kp
    rkp_s[...] = rkp
    dp_s[...] = dp
    drp_s[...] = drp


def kernel(x, slow_W, out_W, ln_g, ln_b):
    x2d = x.reshape(S * B, D)

    qkv = pl.pallas_call(
        _ln_qkv_body,
        grid=(S * B // ROW_BLK,),
        in_specs=[
            pl.BlockSpec((ROW_BLK, D), lambda i: (i, 0)),
            pl.BlockSpec((D, H * E), lambda i: (0, 0)),
            pl.BlockSpec((1, D), lambda i: (0, 0)),
            pl.BlockSpec((1, D), lambda i: (0, 0)),
        ],
        out_specs=pl.BlockSpec((ROW_BLK, H * E), lambda i: (i, 0)),
        out_shape=jax.ShapeDtypeStruct((S * B, H * E), jnp.float32),
        compiler_params=pltpu.CompilerParams(
            dimension_semantics=("parallel",)),
    )(x2d, slow_W.T, ln_g.reshape(1, D), ln_b.reshape(1, D))

    qkv_t = qkv.reshape(S, B, H, E).transpose(0, 3, 2, 1).reshape(S, E, G)

    f32 = jnp.float32
    hs = pl.pallas_call(
        _scan_body,
        grid=(2, S // T_BLK),
        in_specs=[pl.BlockSpec((T_BLK, E, GC), lambda c, s: (s, 0, c))],
        out_specs=pl.BlockSpec((T_BLK, DH, GC), lambda c, s: (s, 0, c)),
        out_shape=jax.ShapeDtypeStruct((S, DH, G), f32),
        scratch_shapes=[
            pltpu.VMEM((DH, DH // 8, 8, GC), f32),
            pltpu.VMEM((DH, DH // 8, 8, GC), f32),
            pltpu.VMEM((DH, GC), f32),
            pltpu.VMEM((T_BLK, DH, GC), f32),
            pltpu.VMEM((T_BLK, DH, GC), f32),
            pltpu.VMEM((T_BLK, DH, GC), f32),
            pltpu.VMEM((T_BLK, 8, GC), f32),
            pltpu.VMEM((T_BLK, 8, GC), f32),
            pltpu.VMEM((DH, 8, GC), f32),
            pltpu.VMEM((DH, 8, GC), f32),
            pltpu.VMEM((DH // 8, 8, GC), f32),
            pltpu.VMEM((DH // 8, 8, GC), f32),
        ],
        compiler_params=pltpu.CompilerParams(
            dimension_semantics=("parallel", "arbitrary")),
    )(qkv_t)

    hs2d = (hs.reshape(S, DH, H, B).transpose(0, 3, 2, 1)
            .reshape(S * B, H * DH))

    y = pl.pallas_call(
        _out_body,
        grid=(S * B // ROW_BLK,),
        in_specs=[
            pl.BlockSpec((ROW_BLK, H * DH), lambda i: (i, 0)),
            pl.BlockSpec((H * DH, D), lambda i: (0, 0)),
            pl.BlockSpec((ROW_BLK, D), lambda i: (i, 0)),
        ],
        out_specs=pl.BlockSpec((ROW_BLK, D), lambda i: (i, 0)),
        out_shape=jax.ShapeDtypeStruct((S * B, D), jnp.float32),
        compiler_params=pltpu.CompilerParams(
            dimension_semantics=("parallel",)),
    )(hs2d, out_W.T, x2d)

    return y.reshape(S, B, D)


# speedup vs baseline: 19.6177x; 1.3565x over previous
import functools

import jax
import jax.numpy as jnp
from jax.experimental import pallas as pl
from jax.experimental.pallas import tpu as pltpu

S, B, D, H, DH = 512, 32, 256, 8, 32
E = 5 * DH + 2
G = B * H
GC = G // 2
LN_EPS = 1e-5

T_BLK = 64
ROW_BLK = 256


def _ln_qkv_body(x_ref, w_ref, g_ref, b_ref, out_ref):
    x = x_ref[...]
    mu = jnp.mean(x, axis=1, keepdims=True)
    xc = x - mu
    var = jnp.mean(xc * xc, axis=1, keepdims=True)
    o = xc * jax.lax.rsqrt(var + LN_EPS) * g_ref[...] + b_ref[...]
    m = jnp.dot(o, w_ref[...], preferred_element_type=jnp.float32)
    parts = []
    for hh in range(H):
        sub = m[:, hh * E:(hh + 1) * E].reshape(ROW_BLK // B, B, E)
        parts.append(jnp.swapaxes(sub, 1, 2))
    out_ref[...] = jnp.concatenate(parts, axis=2)


def _out_body(h_ref, w_ref, x_ref, out_ref):
    blk = h_ref[...]
    parts = []
    for hh in range(H):
        sub = blk[:, :, hh * B:(hh + 1) * B]
        parts.append(jnp.swapaxes(sub, 1, 2))
    hs_tile = jnp.concatenate(parts, axis=2).reshape(ROW_BLK, H * DH)
    out_ref[...] = x_ref[...] + jnp.dot(
        hs_tile, w_ref[...], preferred_element_type=jnp.float32)


def _rep8(x2d):
    return jnp.broadcast_to(x2d.reshape(DH, 1, GC), (DH, 8, GC))


def _dot_rep(a2d, b2d):
    p = (a2d * b2d).reshape(DH // 8, 8, GC)
    s = p[0] + p[1] + p[2] + p[3]
    tot = jnp.sum(s, axis=0, keepdims=True)
    return jnp.broadcast_to(tot, (8, GC))


def _scan_body(qkv_ref, out_ref,
               W_s, R_s, h_s, qa_s, ka_s, rka_s, bb_s, rbb_s,
               kp_s, rkp_s, dp_s, drp_s):
    sb = pl.program_id(1)

    @pl.when(sb == 0)
    def _init():
        W_s[...] = jnp.zeros_like(W_s)
        R_s[...] = jnp.zeros_like(R_s)
        h_s[...] = jnp.zeros_like(h_s)
        kp_s[...] = jnp.zeros_like(kp_s)
        rkp_s[...] = jnp.zeros_like(rkp_s)
        dp_s[...] = jnp.zeros_like(dp_s)
        drp_s[...] = jnp.zeros_like(drp_s)

    q = qkv_ref[:, 0 * DH:1 * DH, :]
    qa = jnp.where(q > 0, q + 1.0, jnp.exp(q))
    qa_s[...] = qa / jnp.sum(qa, axis=1, keepdims=True)
    k = qkv_ref[:, 1 * DH:2 * DH, :]
    ka = jnp.where(k > 0, k + 1.0, jnp.exp(k))
    ka_s[...] = ka / jnp.sum(ka, axis=1, keepdims=True)
    rk = qkv_ref[:, 3 * DH:4 * DH, :]
    rk = rk - jnp.max(rk, axis=1, keepdims=True)
    erk = jnp.exp(rk)
    rka_s[...] = erk / jnp.sum(erk, axis=1, keepdims=True)
    bb_s[...] = jnp.broadcast_to(
        jax.nn.sigmoid(qkv_ref[:, 5 * DH, :])[:, None, :], (T_BLK, 8, GC))
    rbb_s[...] = jnp.broadcast_to(
        jax.nn.sigmoid(qkv_ref[:, 5 * DH + 1, :])[:, None, :],
        (T_BLK, 8, GC))

    def step(t, carry):
        h, kp, rkp, dp, drp = carry
        kt = ka_s[t]
        qt = qa_s[t]
        k8 = _rep8(kt)
        q8 = _rep8(qt)

        accv = [jnp.zeros((DH // 8, 8, GC), jnp.float32)] * 2
        accz = [jnp.zeros((DH // 8, 8, GC), jnp.float32)] * 2
        for j in range(DH):
            w = W_s[j] + kp[j][None] * dp
            W_s[j] = w
            accv[j % 2] = accv[j % 2] + w * k8[j][None]
            accz[j % 2] = accz[j % 2] + w * q8[j][None]
        v_old = accv[0] + accv[1]
        z_old = accz[0] + accz[1]

        vt = qkv_ref[t, 2 * DH:3 * DH, :].reshape(DH // 8, 8, GC)
        bt = bb_s[t][None]
        d = bt * (vt - v_old)
        kq = _dot_rep(kt, qt)
        z = z_old + d * kq[None]

        m = jnp.max(h, axis=(0, 1), keepdims=True)
        eh = jnp.exp(h - m)
        qr = eh / jnp.sum(eh, axis=(0, 1), keepdims=True)
        qr2 = qr.reshape(DH, GC)
        qr8 = _rep8(qr2)
        rkt = rka_s[t]
        rk8 = _rep8(rkt)

        accvr = [jnp.zeros((DH // 8, 8, GC), jnp.float32)] * 2
        acch = [jnp.zeros((DH // 8, 8, GC), jnp.float32)] * 2
        for j in range(DH):
            r = R_s[j] + rkp[j][None] * drp
            R_s[j] = r
            accvr[j % 2] = accvr[j % 2] + r * rk8[j][None]
            acch[j % 2] = acch[j % 2] + r * qr8[j][None]
        v_old_r = accvr[0] + accvr[1]
        h_old = acch[0] + acch[1]

        rvt = qkv_ref[t, 4 * DH:5 * DH, :].reshape(DH // 8, 8, GC)
        rbt = rbb_s[t][None]
        dr = rbt * (rvt - v_old_r)
        rkq = _dot_rep(rkt, qr2)
        h = z + h_old + dr * rkq[None]

        out_ref[t] = h.reshape(DH, GC)
        return h, k8, rk8, d, dr

    h, kp, rkp, dp, drp = jax.lax.fori_loop(
        0, T_BLK, step,
        (h_s[...].reshape(DH // 8, 8, GC), kp_s[...], rkp_s[...],
         dp_s[...], drp_s[...]))
    h_s[...] = h.reshape(DH, GC)
    kp_s[...] = kp
    rkp_s[...] = rkp
    dp_s[...] = dp
    drp_s[...] = drp


def kernel(x, slow_W, out_W, ln_g, ln_b):
    x2d = x.reshape(S * B, D)

    qkv_t = pl.pallas_call(
        _ln_qkv_body,
        grid=(S * B // ROW_BLK,),
        in_specs=[
            pl.BlockSpec((ROW_BLK, D), lambda i: (i, 0)),
            pl.BlockSpec((D, H * E), lambda i: (0, 0)),
            pl.BlockSpec((1, D), lambda i: (0, 0)),
            pl.BlockSpec((1, D), lambda i: (0, 0)),
        ],
        out_specs=pl.BlockSpec((ROW_BLK // B, E, G), lambda i: (i, 0, 0)),
        out_shape=jax.ShapeDtypeStruct((S, E, G), jnp.float32),
        compiler_params=pltpu.CompilerParams(
            dimension_semantics=("parallel",)),
    )(x2d, slow_W.T, ln_g.reshape(1, D), ln_b.reshape(1, D))

    f32 = jnp.float32
    hs = pl.pallas_call(
        _scan_body,
        grid=(2, S // T_BLK),
        in_specs=[pl.BlockSpec((T_BLK, E, GC), lambda c, s: (s, 0, c))],
        out_specs=pl.BlockSpec((T_BLK, DH, GC), lambda c, s: (s, 0, c)),
        out_shape=jax.ShapeDtypeStruct((S, DH, G), f32),
        scratch_shapes=[
            pltpu.VMEM((DH, DH // 8, 8, GC), f32),
            pltpu.VMEM((DH, DH // 8, 8, GC), f32),
            pltpu.VMEM((DH, GC), f32),
            pltpu.VMEM((T_BLK, DH, GC), f32),
            pltpu.VMEM((T_BLK, DH, GC), f32),
            pltpu.VMEM((T_BLK, DH, GC), f32),
            pltpu.VMEM((T_BLK, 8, GC), f32),
            pltpu.VMEM((T_BLK, 8, GC), f32),
            pltpu.VMEM((DH, 8, GC), f32),
            pltpu.VMEM((DH, 8, GC), f32),
            pltpu.VMEM((DH // 8, 8, GC), f32),
            pltpu.VMEM((DH // 8, 8, GC), f32),
        ],
        compiler_params=pltpu.CompilerParams(
            dimension_semantics=("parallel", "arbitrary")),
    )(qkv_t)

    y = pl.pallas_call(
        _out_body,
        grid=(S * B // ROW_BLK,),
        in_specs=[
            pl.BlockSpec((ROW_BLK // B, DH, G), lambda i: (i, 0, 0)),
            pl.BlockSpec((H * DH, D), lambda i: (0, 0)),
            pl.BlockSpec((ROW_BLK, D), lambda i: (i, 0)),
        ],
        out_specs=pl.BlockSpec((ROW_BLK, D), lambda i: (i, 0)),
        out_shape=jax.ShapeDtypeStruct((S * B, D), jnp.float32),
        compiler_params=pltpu.CompilerParams(
            dimension_semantics=("parallel",)),
    )(hs, out_W.T, x2d)

    return y.reshape(S, B, D)


# speedup vs baseline: 21.2253x; 1.0819x over previous
import functools

import jax
import jax.numpy as jnp
from jax.experimental import pallas as pl
from jax.experimental.pallas import tpu as pltpu

S, B, D, H, DH = 512, 32, 256, 8, 32
E = 5 * DH + 2
G = B * H
GC = G // 2
LN_EPS = 1e-5

T_BLK = 64
ROW_BLK = 256


def _ln_qkv_body(x_ref, w_ref, g_ref, b_ref, out_ref):
    x = x_ref[...]
    mu = jnp.mean(x, axis=1, keepdims=True)
    xc = x - mu
    var = jnp.mean(xc * xc, axis=1, keepdims=True)
    o = xc * jax.lax.rsqrt(var + LN_EPS) * g_ref[...] + b_ref[...]
    m = jnp.dot(o, w_ref[...], preferred_element_type=jnp.float32)
    parts = []
    for hh in range(H):
        sub = m[:, hh * E:(hh + 1) * E].reshape(ROW_BLK // B, B, E)
        parts.append(jnp.swapaxes(sub, 1, 2))
    out_ref[...] = jnp.concatenate(parts, axis=2)


def _out_body(h_ref, w_ref, x_ref, out_ref):
    blk = h_ref[...]
    parts = []
    for hh in range(H):
        sub = blk[:, :, hh * B:(hh + 1) * B]
        parts.append(jnp.swapaxes(sub, 1, 2))
    hs_tile = jnp.concatenate(parts, axis=2).reshape(ROW_BLK, H * DH)
    out_ref[...] = x_ref[...] + jnp.dot(
        hs_tile, w_ref[...], preferred_element_type=jnp.float32)


def _rep8(x2d):
    return jnp.broadcast_to(x2d.reshape(DH, 1, GC), (DH, 8, GC))


def _dot_rep(a2d, b2d):
    p = (a2d * b2d).reshape(DH // 8, 8, GC)
    s = p[0] + p[1] + p[2] + p[3]
    tot = jnp.sum(s, axis=0, keepdims=True)
    return jnp.broadcast_to(tot, (8, GC))


def _scan_body(qkv_ref, out_ref,
               W_s, R_s, h_s, qa_s, ka_s, rka_s, bb_s, rbb_s,
               kp_s, rkp_s, dp_s, drp_s):
    sb = pl.program_id(1)

    @pl.when(sb == 0)
    def _init():
        W_s[...] = jnp.zeros_like(W_s)
        R_s[...] = jnp.zeros_like(R_s)
        h_s[...] = jnp.zeros_like(h_s)
        kp_s[...] = jnp.zeros_like(kp_s)
        rkp_s[...] = jnp.zeros_like(rkp_s)
        dp_s[...] = jnp.zeros_like(dp_s)
        drp_s[...] = jnp.zeros_like(drp_s)

    q = qkv_ref[:, 0 * DH:1 * DH, :]
    qa = jnp.where(q > 0, q + 1.0, jnp.exp(q))
    qa_s[...] = qa / jnp.sum(qa, axis=1, keepdims=True)
    k = qkv_ref[:, 1 * DH:2 * DH, :]
    ka = jnp.where(k > 0, k + 1.0, jnp.exp(k))
    ka_s[...] = ka / jnp.sum(ka, axis=1, keepdims=True)
    rk = qkv_ref[:, 3 * DH:4 * DH, :]
    rk = rk - jnp.max(rk, axis=1, keepdims=True)
    erk = jnp.exp(rk)
    rka_s[...] = erk / jnp.sum(erk, axis=1, keepdims=True)
    bb_s[...] = jnp.broadcast_to(
        jax.nn.sigmoid(qkv_ref[:, 5 * DH, :])[:, None, :], (T_BLK, 8, GC))
    rbb_s[...] = jnp.broadcast_to(
        jax.nn.sigmoid(qkv_ref[:, 5 * DH + 1, :])[:, None, :],
        (T_BLK, 8, GC))

    def step(t, carry):
        h, kprev, rkprev, dp, drp = carry
        kv = ka_s[t]
        qv = qa_s[t]

        accv = [jnp.zeros((DH // 8, 8, GC), jnp.float32)] * 2
        accz = [jnp.zeros((DH // 8, 8, GC), jnp.float32)] * 2
        for j in range(DH):
            kbp = jnp.broadcast_to(kprev[j:j + 1, :], (8, GC))
            kb = jnp.broadcast_to(kv[j:j + 1, :], (8, GC))
            qb = jnp.broadcast_to(qv[j:j + 1, :], (8, GC))
            w = W_s[j] + kbp[None] * dp
            W_s[j] = w
            accv[j % 2] = accv[j % 2] + w * kb[None]
            accz[j % 2] = accz[j % 2] + w * qb[None]
        v_old = accv[0] + accv[1]
        z_old = accz[0] + accz[1]

        vt = qkv_ref[t, 2 * DH:3 * DH, :].reshape(DH // 8, 8, GC)
        bt = bb_s[t][None]
        d = bt * (vt - v_old)
        kq = _dot_rep(kv, qv)
        z = z_old + d * kq[None]

        m = jnp.max(h, axis=(0, 1), keepdims=True)
        eh = jnp.exp(h - m)
        qr = eh / jnp.sum(eh, axis=(0, 1), keepdims=True)
        qr2 = qr.reshape(DH, GC)
        rkv = rka_s[t]

        accvr = [jnp.zeros((DH // 8, 8, GC), jnp.float32)] * 2
        acch = [jnp.zeros((DH // 8, 8, GC), jnp.float32)] * 2
        for j in range(DH):
            rkbp = jnp.broadcast_to(rkprev[j:j + 1, :], (8, GC))
            rkb = jnp.broadcast_to(rkv[j:j + 1, :], (8, GC))
            qrb = jnp.broadcast_to(qr2[j:j + 1, :], (8, GC))
            r = R_s[j] + rkbp[None] * drp
            R_s[j] = r
            accvr[j % 2] = accvr[j % 2] + r * rkb[None]
            acch[j % 2] = acch[j % 2] + r * qrb[None]
        v_old_r = accvr[0] + accvr[1]
        h_old = acch[0] + acch[1]

        rvt = qkv_ref[t, 4 * DH:5 * DH, :].reshape(DH // 8, 8, GC)
        rbt = rbb_s[t][None]
        dr = rbt * (rvt - v_old_r)
        rkq = _dot_rep(rkv, qr2)
        h = z + h_old + dr * rkq[None]

        out_ref[t] = h.reshape(DH, GC)
        return h, kv, rkv, d, dr

    h, kprev, rkprev, dp, drp = jax.lax.fori_loop(
        0, T_BLK, step,
        (h_s[...].reshape(DH // 8, 8, GC), kp_s[...], rkp_s[...],
         dp_s[...], drp_s[...]))
    h_s[...] = h.reshape(DH, GC)
    kp_s[...] = kprev
    rkp_s[...] = rkprev
    dp_s[...] = dp
    drp_s[...] = drp


def kernel(x, slow_W, out_W, ln_g, ln_b):
    x2d = x.reshape(S * B, D)

    qkv_t = pl.pallas_call(
        _ln_qkv_body,
        grid=(S * B // ROW_BLK,),
        in_specs=[
            pl.BlockSpec((ROW_BLK, D), lambda i: (i, 0)),
            pl.BlockSpec((D, H * E), lambda i: (0, 0)),
            pl.BlockSpec((1, D), lambda i: (0, 0)),
            pl.BlockSpec((1, D), lambda i: (0, 0)),
        ],
        out_specs=pl.BlockSpec((ROW_BLK // B, E, G), lambda i: (i, 0, 0)),
        out_shape=jax.ShapeDtypeStruct((S, E, G), jnp.float32),
        compiler_params=pltpu.CompilerParams(
            dimension_semantics=("parallel",)),
    )(x2d, slow_W.T, ln_g.reshape(1, D), ln_b.reshape(1, D))

    f32 = jnp.float32
    hs = pl.pallas_call(
        _scan_body,
        grid=(2, S // T_BLK),
        in_specs=[pl.BlockSpec((T_BLK, E, GC), lambda c, s: (s, 0, c))],
        out_specs=pl.BlockSpec((T_BLK, DH, GC), lambda c, s: (s, 0, c)),
        out_shape=jax.ShapeDtypeStruct((S, DH, G), f32),
        scratch_shapes=[
            pltpu.VMEM((DH, DH // 8, 8, GC), f32),
            pltpu.VMEM((DH, DH // 8, 8, GC), f32),
            pltpu.VMEM((DH, GC), f32),
            pltpu.VMEM((T_BLK, DH, GC), f32),
            pltpu.VMEM((T_BLK, DH, GC), f32),
            pltpu.VMEM((T_BLK, DH, GC), f32),
            pltpu.VMEM((T_BLK, 8, GC), f32),
            pltpu.VMEM((T_BLK, 8, GC), f32),
            pltpu.VMEM((DH, GC), f32),
            pltpu.VMEM((DH, GC), f32),
            pltpu.VMEM((DH // 8, 8, GC), f32),
            pltpu.VMEM((DH // 8, 8, GC), f32),
        ],
        compiler_params=pltpu.CompilerParams(
            dimension_semantics=("parallel", "arbitrary")),
    )(qkv_t)

    y = pl.pallas_call(
        _out_body,
        grid=(S * B // ROW_BLK,),
        in_specs=[
            pl.BlockSpec((ROW_BLK // B, DH, G), lambda i: (i, 0, 0)),
            pl.BlockSpec((H * DH, D), lambda i: (0, 0)),
            pl.BlockSpec((ROW_BLK, D), lambda i: (i, 0)),
        ],
        out_specs=pl.BlockSpec((ROW_BLK, D), lambda i: (i, 0)),
        out_shape=jax.ShapeDtypeStruct((S * B, D), jnp.float32),
        compiler_params=pltpu.CompilerParams(
            dimension_semantics=("parallel",)),
    )(hs, out_W.T, x2d)

    return y.reshape(S, B, D)


# speedup vs baseline: 22.5527x; 1.0625x over previous
import functools

import jax
import jax.numpy as jnp
from jax.experimental import pallas as pl
from jax.experimental.pallas import tpu as pltpu

S, B, D, H, DH = 512, 32, 256, 8, 32
E = 5 * DH + 2
G = B * H
GC = G // 2
LN_EPS = 1e-5

T_BLK = 64
ROW_BLK = 512


def _ln_qkv_body(x_ref, w_ref, g_ref, b_ref, out_ref):
    x = x_ref[...]
    mu = jnp.mean(x, axis=1, keepdims=True)
    xc = x - mu
    var = jnp.mean(xc * xc, axis=1, keepdims=True)
    o = xc * jax.lax.rsqrt(var + LN_EPS) * g_ref[...] + b_ref[...]
    m = jnp.dot(o, w_ref[...], preferred_element_type=jnp.float32)
    parts = []
    for hh in range(H):
        sub = m[:, hh * E:(hh + 1) * E].reshape(ROW_BLK // B, B, E)
        parts.append(jnp.swapaxes(sub, 1, 2))
    out_ref[...] = jnp.concatenate(parts, axis=2)


def _out_body(h_ref, w_ref, x_ref, out_ref):
    blk = h_ref[...]
    parts = []
    for hh in range(H):
        sub = blk[:, :, hh * B:(hh + 1) * B]
        parts.append(jnp.swapaxes(sub, 1, 2))
    hs_tile = jnp.concatenate(parts, axis=2).reshape(ROW_BLK, H * DH)
    out_ref[...] = x_ref[...] + jnp.dot(
        hs_tile, w_ref[...], preferred_element_type=jnp.float32)


def _rep8(x2d):
    return jnp.broadcast_to(x2d.reshape(DH, 1, GC), (DH, 8, GC))


def _dot_rep(a2d, b2d):
    p = (a2d * b2d).reshape(DH // 8, 8, GC)
    s = p[0] + p[1] + p[2] + p[3]
    tot = jnp.sum(s, axis=0, keepdims=True)
    return jnp.broadcast_to(tot, (8, GC))


def _scan_body(qkv_ref, out_ref,
               W_s, R_s, h_s, qa_s, ka_s, rka_s, bb_s, rbb_s,
               kp_s, rkp_s, dp_s, drp_s):
    sb = pl.program_id(1)

    @pl.when(sb == 0)
    def _init():
        W_s[...] = jnp.zeros_like(W_s)
        R_s[...] = jnp.zeros_like(R_s)
        h_s[...] = jnp.zeros_like(h_s)
        kp_s[...] = jnp.zeros_like(kp_s)
        rkp_s[...] = jnp.zeros_like(rkp_s)
        dp_s[...] = jnp.zeros_like(dp_s)
        drp_s[...] = jnp.zeros_like(drp_s)

    q = qkv_ref[:, 0 * DH:1 * DH, :]
    qa = jnp.where(q > 0, q + 1.0, jnp.exp(q))
    qa_s[...] = qa / jnp.sum(qa, axis=1, keepdims=True)
    k = qkv_ref[:, 1 * DH:2 * DH, :]
    ka = jnp.where(k > 0, k + 1.0, jnp.exp(k))
    ka_s[...] = ka / jnp.sum(ka, axis=1, keepdims=True)
    rk = qkv_ref[:, 3 * DH:4 * DH, :]
    rk = rk - jnp.max(rk, axis=1, keepdims=True)
    erk = jnp.exp(rk)
    rka_s[...] = erk / jnp.sum(erk, axis=1, keepdims=True)
    bb_s[...] = jnp.broadcast_to(
        jax.nn.sigmoid(qkv_ref[:, 5 * DH, :])[:, None, :], (T_BLK, 8, GC))
    rbb_s[...] = jnp.broadcast_to(
        jax.nn.sigmoid(qkv_ref[:, 5 * DH + 1, :])[:, None, :],
        (T_BLK, 8, GC))

    def step(t, carry):
        h, kprev, rkprev, dp, drp = carry
        kv = ka_s[t]
        qv = qa_s[t]

        accv = [jnp.zeros((DH // 8, 8, GC), jnp.float32)] * 2
        accz = [jnp.zeros((DH // 8, 8, GC), jnp.float32)] * 2
        for j in range(DH):
            kbp = jnp.broadcast_to(kprev[j:j + 1, :], (8, GC))
            kb = jnp.broadcast_to(kv[j:j + 1, :], (8, GC))
            qb = jnp.broadcast_to(qv[j:j + 1, :], (8, GC))
            w = W_s[j] + kbp[None] * dp
            W_s[j] = w
            accv[j % 2] = accv[j % 2] + w * kb[None]
            accz[j % 2] = accz[j % 2] + w * qb[None]
        v_old = accv[0] + accv[1]
        z_old = accz[0] + accz[1]

        vt = qkv_ref[t, 2 * DH:3 * DH, :].reshape(DH // 8, 8, GC)
        bt = bb_s[t][None]
        d = bt * (vt - v_old)
        kq = _dot_rep(kv, qv)
        z = z_old + d * kq[None]

        m = jnp.max(h, axis=(0, 1), keepdims=True)
        eh = jnp.exp(h - m)
        qr = eh / jnp.sum(eh, axis=(0, 1), keepdims=True)
        qr2 = qr.reshape(DH, GC)
        rkv = rka_s[t]

        accvr = [jnp.zeros((DH // 8, 8, GC), jnp.float32)] * 2
        acch = [jnp.zeros((DH // 8, 8, GC), jnp.float32)] * 2
        for j in range(DH):
            rkbp = jnp.broadcast_to(rkprev[j:j + 1, :], (8, GC))
            rkb = jnp.broadcast_to(rkv[j:j + 1, :], (8, GC))
            qrb = jnp.broadcast_to(qr2[j:j + 1, :], (8, GC))
            r = R_s[j] + rkbp[None] * drp
            R_s[j] = r
            accvr[j % 2] = accvr[j % 2] + r * rkb[None]
            acch[j % 2] = acch[j % 2] + r * qrb[None]
        v_old_r = accvr[0] + accvr[1]
        h_old = acch[0] + acch[1]

        rvt = qkv_ref[t, 4 * DH:5 * DH, :].reshape(DH // 8, 8, GC)
        rbt = rbb_s[t][None]
        dr = rbt * (rvt - v_old_r)
        rkq = _dot_rep(rkv, qr2)
        h = z + h_old + dr * rkq[None]

        out_ref[t] = h.reshape(DH, GC)
        return h, kv, rkv, d, dr

    h, kprev, rkprev, dp, drp = jax.lax.fori_loop(
        0, T_BLK, step,
        (h_s[...].reshape(DH // 8, 8, GC), kp_s[...], rkp_s[...],
         dp_s[...], drp_s[...]))
    h_s[...] = h.reshape(DH, GC)
    kp_s[...] = kprev
    rkp_s[...] = rkprev
    dp_s[...] = dp
    drp_s[...] = drp


def kernel(x, slow_W, out_W, ln_g, ln_b):
    x2d = x.reshape(S * B, D)

    qkv_t = pl.pallas_call(
        _ln_qkv_body,
        grid=(S * B // ROW_BLK,),
        in_specs=[
            pl.BlockSpec((ROW_BLK, D), lambda i: (i, 0)),
            pl.BlockSpec((D, H * E), lambda i: (0, 0)),
            pl.BlockSpec((1, D), lambda i: (0, 0)),
            pl.BlockSpec((1, D), lambda i: (0, 0)),
        ],
        out_specs=pl.BlockSpec((ROW_BLK // B, E, G), lambda i: (i, 0, 0)),
        out_shape=jax.ShapeDtypeStruct((S, E, G), jnp.float32),
        compiler_params=pltpu.CompilerParams(
            dimension_semantics=("parallel",)),
    )(x2d, slow_W.T, ln_g.reshape(1, D), ln_b.reshape(1, D))

    f32 = jnp.float32
    hs = pl.pallas_call(
        _scan_body,
        grid=(2, S // T_BLK),
        in_specs=[pl.BlockSpec((T_BLK, E, GC), lambda c, s: (s, 0, c))],
        out_specs=pl.BlockSpec((T_BLK, DH, GC), lambda c, s: (s, 0, c)),
        out_shape=jax.ShapeDtypeStruct((S, DH, G), f32),
        scratch_shapes=[
            pltpu.VMEM((DH, DH // 8, 8, GC), f32),
            pltpu.VMEM((DH, DH // 8, 8, GC), f32),
            pltpu.VMEM((DH, GC), f32),
            pltpu.VMEM((T_BLK, DH, GC), f32),
            pltpu.VMEM((T_BLK, DH, GC), f32),
            pltpu.VMEM((T_BLK, DH, GC), f32),
            pltpu.VMEM((T_BLK, 8, GC), f32),
            pltpu.VMEM((T_BLK, 8, GC), f32),
            pltpu.VMEM((DH, GC), f32),
            pltpu.VMEM((DH, GC), f32),
            pltpu.VMEM((DH // 8, 8, GC), f32),
            pltpu.VMEM((DH // 8, 8, GC), f32),
        ],
        compiler_params=pltpu.CompilerParams(
            dimension_semantics=("parallel", "arbitrary")),
    )(qkv_t)

    y = pl.pallas_call(
        _out_body,
        grid=(S * B // ROW_BLK,),
        in_specs=[
            pl.BlockSpec((ROW_BLK // B, DH, G), lambda i: (i, 0, 0)),
            pl.BlockSpec((H * DH, D), lambda i: (0, 0)),
            pl.BlockSpec((ROW_BLK, D), lambda i: (i, 0)),
        ],
        out_specs=pl.BlockSpec((ROW_BLK, D), lambda i: (i, 0)),
        out_shape=jax.ShapeDtypeStruct((S * B, D), jnp.float32),
        compiler_params=pltpu.CompilerParams(
            dimension_semantics=("parallel",)),
    )(hs, out_W.T, x2d)

    return y.reshape(S, B, D)
